# baseline (speedup 1.0000x reference)
import functools

import jax
import jax.numpy as jnp
from jax import lax
from jax.experimental import pallas as pl
from jax.experimental.pallas import tpu as pltpu
from jax.experimental.pallas import tpu_sc as plsc

N_NODES = 100000
EMB_DIM = 64

NUM_CORES = 2
NUM_SUBCORES = 16
NW = NUM_CORES * NUM_SUBCORES

B_PER_W = 3200
B_PAD = NW * B_PER_W
CHUNK = 800
NCHUNKS = B_PER_W // CHUNK


def _gather_body(idx_hbm, table_hbm, out_hbm, idx_v, buf0, buf1, sem0, sem1):
    wid = lax.axis_index("s") * NUM_CORES + lax.axis_index("c")
    base = wid * B_PER_W
    pltpu.sync_copy(idx_hbm.at[pl.ds(base, B_PER_W)], idx_v)
    bufs = (buf0, buf1)
    sems = (sem0, sem1)
    copies = [None, None]
    copies[0] = pltpu.async_copy(
        table_hbm.at[idx_v.at[pl.ds(0, CHUNK)]], buf0, sem0)
    for c in range(NCHUNKS):
        cur = c % 2
        nxt = (c + 1) % 2
        if c + 1 < NCHUNKS:
            copies[nxt] = pltpu.async_copy(
                table_hbm.at[idx_v.at[pl.ds((c + 1) * CHUNK, CHUNK)]],
                bufs[nxt], sems[nxt])
        copies[cur].wait()
        pltpu.sync_copy(bufs[cur], out_hbm.at[pl.ds(base + c * CHUNK, CHUNK)])


@functools.partial(
    pl.kernel,
    mesh=plsc.VectorSubcoreMesh(core_axis_name="c", subcore_axis_name="s"),
    out_type=jax.ShapeDtypeStruct((B_PAD, EMB_DIM), jnp.float32),
    scratch_types=[
        pltpu.VMEM((B_PER_W,), jnp.int32),
        pltpu.VMEM((CHUNK, EMB_DIM), jnp.float32),
        pltpu.VMEM((CHUNK, EMB_DIM), jnp.float32),
        pltpu.SemaphoreType.DMA,
        pltpu.SemaphoreType.DMA,
    ],
    compiler_params=pltpu.CompilerParams(use_tc_tiling_on_sc=False),
)
def _sc_gather(idx_hbm, table_hbm, out_hbm, idx_v, buf0, buf1, sem0, sem1):
    _gather_body(idx_hbm, table_hbm, out_hbm, idx_v, buf0, buf1, sem0, sem1)


def kernel(x, table):
    idx = x.reshape(-1)
    idx_pad = jnp.pad(idx, (0, B_PAD - N_NODES))
    out = _sc_gather(idx_pad, table)
    return out[:N_NODES]


# speedup vs baseline: 1.1187x; 1.1187x over previous
import functools

import jax
import jax.numpy as jnp
from jax import lax
from jax.experimental import pallas as pl
from jax.experimental.pallas import tpu as pltpu
from jax.experimental.pallas import tpu_sc as plsc

N_NODES = 100000
EMB_DIM = 64

NUM_CORES = 2
NUM_SUBCORES = 16
NW = NUM_CORES * NUM_SUBCORES

B_PER_W = 3128
CHUNKS = (800, 800, 800, 728)
CHUNK_MAX = 800


def _gather_body(idx_hbm, table_hbm, out_hbm, idx_v, buf0, buf1, sem0, sem1):
    wid = lax.axis_index("s") * NUM_CORES + lax.axis_index("c")
    base = jnp.minimum(wid * B_PER_W, N_NODES - B_PER_W)
    pltpu.sync_copy(idx_hbm.at[pl.ds(base, B_PER_W)], idx_v)
    bufs = (buf0, buf1)
    sems = (sem0, sem1)
    offs = [0]
    for c in CHUNKS:
        offs.append(offs[-1] + c)
    copies = [None, None]
    copies[0] = pltpu.async_copy(
        table_hbm.at[idx_v.at[pl.ds(0, CHUNKS[0])]],
        buf0.at[pl.ds(0, CHUNKS[0])], sem0)
    for c in range(len(CHUNKS)):
        cur = c % 2
        nxt = (c + 1) % 2
        if c + 1 < len(CHUNKS):
            sz = CHUNKS[c + 1]
            copies[nxt] = pltpu.async_copy(
                table_hbm.at[idx_v.at[pl.ds(offs[c + 1], sz)]],
                bufs[nxt].at[pl.ds(0, sz)], sems[nxt])
        copies[cur].wait()
        pltpu.sync_copy(bufs[cur].at[pl.ds(0, CHUNKS[c])],
                        out_hbm.at[pl.ds(base + offs[c], CHUNKS[c])])


@functools.partial(
    pl.kernel,
    mesh=plsc.VectorSubcoreMesh(core_axis_name="c", subcore_axis_name="s"),
    out_type=jax.ShapeDtypeStruct((N_NODES, EMB_DIM), jnp.float32),
    scratch_types=[
        pltpu.VMEM((B_PER_W,), jnp.int32),
        pltpu.VMEM((CHUNK_MAX, EMB_DIM), jnp.float32),
        pltpu.VMEM((CHUNK_MAX, EMB_DIM), jnp.float32),
        pltpu.SemaphoreType.DMA,
        pltpu.SemaphoreType.DMA,
    ],
    compiler_params=pltpu.CompilerParams(use_tc_tiling_on_sc=False),
)
def _sc_gather(idx_hbm, table_hbm, out_hbm, idx_v, buf0, buf1, sem0, sem1):
    _gather_body(idx_hbm, table_hbm, out_hbm, idx_v, buf0, buf1, sem0, sem1)


def kernel(x, table):
    idx = x.reshape(-1)
    return _sc_gather(idx, table)


# speedup vs baseline: 1.2216x; 1.0920x over previous
import functools

import jax
import jax.numpy as jnp
from jax import lax
from jax.experimental import pallas as pl
from jax.experimental.pallas import tpu as pltpu
from jax.experimental.pallas import tpu_sc as plsc

N_NODES = 100000
EMB_DIM = 64
N_ROWS = 1000001

TBLK = 2048
TGRID = -(-N_ROWS // TBLK)
ROWS_PAD = TGRID * TBLK
FLAT = ROWS_PAD * EMB_DIM

NUM_CORES = 2
NUM_SUBCORES = 16
NW = NUM_CORES * NUM_SUBCORES
B_PER_W = 3128
CHUNKS = (800, 800, 800, 728)
CHUNK_MAX = 800


def _transpose_body(src_ref, dst_ref):
    y = src_ref[...].T.reshape(TBLK // 2, 2, EMB_DIM)
    dst_ref[:, 0:EMB_DIM] = y[:, 0, :]
    dst_ref[:, EMB_DIM:2 * EMB_DIM] = y[:, 1, :]


_tc_transpose = pl.pallas_call(
    _transpose_body,
    grid=(TGRID,),
    in_specs=[pl.BlockSpec((EMB_DIM, TBLK), lambda k: (0, k))],
    out_specs=pl.BlockSpec((TBLK // 2, 2 * EMB_DIM), lambda k: (k, 0)),
    out_shape=jax.ShapeDtypeStruct((FLAT // 128, 128), jnp.float32),
)


def _gather_body(idx_hbm, table_hbm, out_hbm, idx_v, buf0, buf1, sem0, sem1):
    wid = lax.axis_index("s") * NUM_CORES + lax.axis_index("c")
    base = jnp.minimum(wid * B_PER_W, N_NODES - B_PER_W)
    pltpu.sync_copy(idx_hbm.at[pl.ds(base, B_PER_W)], idx_v)
    bufs = (buf0, buf1)
    sems = (sem0, sem1)
    offs = [0]
    for c in CHUNKS:
        offs.append(offs[-1] + c)
    copies = [None, None]
    copies[0] = pltpu.async_copy(
        table_hbm.at[idx_v.at[pl.ds(0, CHUNKS[0])]],
        buf0.at[pl.ds(0, CHUNKS[0])], sem0)
    for c in range(len(CHUNKS)):
        cur = c % 2
        nxt = (c + 1) % 2
        if c + 1 < len(CHUNKS):
            sz = CHUNKS[c + 1]
            copies[nxt] = pltpu.async_copy(
                table_hbm.at[idx_v.at[pl.ds(offs[c + 1], sz)]],
                bufs[nxt].at[pl.ds(0, sz)], sems[nxt])
        copies[cur].wait()
        pltpu.sync_copy(bufs[cur].at[pl.ds(0, CHUNKS[c])],
                        out_hbm.at[pl.ds(base + offs[c], CHUNKS[c])])


@functools.partial(
    pl.kernel,
    mesh=plsc.VectorSubcoreMesh(core_axis_name="c", subcore_axis_name="s"),
    out_type=jax.ShapeDtypeStruct((N_NODES, EMB_DIM), jnp.float32),
    scratch_types=[
        pltpu.VMEM((B_PER_W,), jnp.int32),
        pltpu.VMEM((CHUNK_MAX, EMB_DIM), jnp.float32),
        pltpu.VMEM((CHUNK_MAX, EMB_DIM), jnp.float32),
        pltpu.SemaphoreType.DMA,
        pltpu.SemaphoreType.DMA,
    ],
    compiler_params=pltpu.CompilerParams(use_tc_tiling_on_sc=False),
)
def _sc_gather(idx_hbm, table_hbm, out_hbm, idx_v, buf0, buf1, sem0, sem1):
    _gather_body(idx_hbm, table_hbm, out_hbm, idx_v, buf0, buf1, sem0, sem1)


def kernel(x, table):
    idx = x.reshape(-1)
    flat = _tc_transpose(table.T)
    tbl = flat.reshape(ROWS_PAD, EMB_DIM)
    return _sc_gather(idx, tbl)


# speedup vs baseline: 2.1198x; 1.7352x over previous
import functools

import jax
import jax.numpy as jnp
from jax import lax
from jax.experimental import pallas as pl
from jax.experimental.pallas import tpu as pltpu
from jax.experimental.pallas import tpu_sc as plsc

N_NODES = 100000
EMB_DIM = 64
N_ROWS = 1000001

TBLK = 4096
TGRID = -(-N_ROWS // TBLK)
ROWS_PAD = TGRID * TBLK
FLAT = ROWS_PAD * EMB_DIM
HALF_SHIFT = (TBLK // 2).bit_length() - 1

NUM_CORES = 2
NUM_SUBCORES = 16
NW = NUM_CORES * NUM_SUBCORES
B_PER_W = 3128
CHUNKS = (800, 800, 800, 728)
CHUNK_MAX = 800


def _transpose_body(src_ref, dst_ref):
    x = src_ref[...]
    xx = jnp.concatenate([x[:, :TBLK // 2], x[:, TBLK // 2:]], axis=0)
    dst_ref[...] = xx.T


_tc_transpose = pl.pallas_call(
    _transpose_body,
    grid=(TGRID,),
    in_specs=[pl.BlockSpec((EMB_DIM, TBLK), lambda k: (0, k))],
    out_specs=pl.BlockSpec((TBLK // 2, 2 * EMB_DIM), lambda k: (k, 0)),
    out_shape=jax.ShapeDtypeStruct((FLAT // 128, 128), jnp.float32),
)


def _gather_body(idx_hbm, table_hbm, out_hbm, idx_v, buf0, buf1, sem0, sem1):
    wid = lax.axis_index("s") * NUM_CORES + lax.axis_index("c")
    base = jnp.minimum(wid * B_PER_W, N_NODES - B_PER_W)
    pltpu.sync_copy(idx_hbm.at[pl.ds(base, B_PER_W)], idx_v)
    bufs = (buf0, buf1)
    sems = (sem0, sem1)
    offs = [0]
    for c in CHUNKS:
        offs.append(offs[-1] + c)
    copies = [None, None]
    copies[0] = pltpu.async_copy(
        table_hbm.at[idx_v.at[pl.ds(0, CHUNKS[0])]],
        buf0.at[pl.ds(0, CHUNKS[0])], sem0)
    for c in range(len(CHUNKS)):
        cur = c % 2
        nxt = (c + 1) % 2
        if c + 1 < len(CHUNKS):
            sz = CHUNKS[c + 1]
            copies[nxt] = pltpu.async_copy(
                table_hbm.at[idx_v.at[pl.ds(offs[c + 1], sz)]],
                bufs[nxt].at[pl.ds(0, sz)], sems[nxt])
        copies[cur].wait()
        pltpu.sync_copy(bufs[cur].at[pl.ds(0, CHUNKS[c])],
                        out_hbm.at[pl.ds(base + offs[c], CHUNKS[c])])


@functools.partial(
    pl.kernel,
    mesh=plsc.VectorSubcoreMesh(core_axis_name="c", subcore_axis_name="s"),
    out_type=jax.ShapeDtypeStruct((N_NODES, EMB_DIM), jnp.float32),
    scratch_types=[
        pltpu.VMEM((B_PER_W,), jnp.int32),
        pltpu.VMEM((CHUNK_MAX, EMB_DIM), jnp.float32),
        pltpu.VMEM((CHUNK_MAX, EMB_DIM), jnp.float32),
        pltpu.SemaphoreType.DMA,
        pltpu.SemaphoreType.DMA,
    ],
    compiler_params=pltpu.CompilerParams(use_tc_tiling_on_sc=False),
)
def _sc_gather(idx_hbm, table_hbm, out_hbm, idx_v, buf0, buf1, sem0, sem1):
    _gather_body(idx_hbm, table_hbm, out_hbm, idx_v, buf0, buf1, sem0, sem1)


def kernel(x, table):
    idx = x.reshape(-1)
    slot = ((idx & ~(TBLK - 1)) + ((idx & (TBLK // 2 - 1)) << 1)
            + ((idx >> HALF_SHIFT) & 1))
    flat = _tc_transpose(table.T)
    tbl = flat.reshape(ROWS_PAD, EMB_DIM)
    return _sc_gather(slot, tbl)


# speedup vs baseline: 2.4796x; 1.1698x over previous
import functools

import jax
import jax.numpy as jnp
from jax import lax
from jax.experimental import pallas as pl
from jax.experimental.pallas import tpu as pltpu
from jax.experimental.pallas import tpu_sc as plsc

N_NODES = 100000
EMB_DIM = 64
N_ROWS = 1000001

TBLK = 8192
TGRID = -(-N_ROWS // TBLK)
ROWS_PAD = TGRID * TBLK
FLAT = ROWS_PAD * EMB_DIM
HALF_SHIFT = (TBLK // 2).bit_length() - 1

NUM_CORES = 2
NUM_SUBCORES = 16
NW = NUM_CORES * NUM_SUBCORES
B_PER_W = 3136
HB_PER_W = B_PER_W // 2
HCHUNKS = (392, 392, 392, 392)
CHUNK_MAX = 392


def _transpose_body(src_ref, dst_ref):
    x = src_ref[...]
    xx = jnp.concatenate([x[:, :TBLK // 2], x[:, TBLK // 2:]], axis=0)
    dst_ref[...] = xx.T


_tc_transpose = pl.pallas_call(
    _transpose_body,
    grid=(TGRID,),
    in_specs=[pl.BlockSpec((EMB_DIM, TBLK), lambda k: (0, k))],
    out_specs=pl.BlockSpec((TBLK // 2, 2 * EMB_DIM), lambda k: (k, 0)),
    out_shape=jax.ShapeDtypeStruct((FLAT // 128, 128), jnp.float32),
)


def _gather_body(idx_hbm, table_hbm, out_hbm, idx_v, buf0, buf1, sem0, sem1):
    wid = lax.axis_index("s") * NUM_CORES + lax.axis_index("c")
    base = jnp.minimum(wid * B_PER_W, N_NODES - B_PER_W)
    h = pl.multiple_of(base // 2, 8)
    pltpu.sync_copy(idx_hbm.at[pl.ds(h, HB_PER_W)], idx_v.at[pl.ds(0, HB_PER_W)])
    pltpu.sync_copy(idx_hbm.at[pl.ds(N_NODES // 2 + h, HB_PER_W)],
                    idx_v.at[pl.ds(HB_PER_W, HB_PER_W)])
    bufs = (buf0, buf1)
    sems = (sem0, sem1)
    offs = [0]
    for c in HCHUNKS:
        offs.append(offs[-1] + c)
    njobs = 2 * len(HCHUNKS)

    def job(j):
        par = j % 2
        c = j // 2
        return par, offs[c], HCHUNKS[c]

    copies = [None, None]

    def start(j, slot):
        par, off, sz = job(j)
        copies[slot] = pltpu.async_copy(
            table_hbm.at[idx_v.at[pl.ds(par * HB_PER_W + off, sz)]],
            bufs[slot].at[pl.ds(0, sz)], sems[slot])

    start(0, 0)
    for j in range(njobs):
        cur = j % 2
        if j + 1 < njobs:
            start(j + 1, (j + 1) % 2)
        par, off, sz = job(j)
        copies[cur].wait()
        pltpu.sync_copy(
            bufs[cur].at[pl.ds(0, sz)],
            out_hbm.at[pl.ds(h + off, sz),
                       pl.ds(par * EMB_DIM, EMB_DIM)])


@functools.partial(
    pl.kernel,
    mesh=plsc.VectorSubcoreMesh(core_axis_name="c", subcore_axis_name="s"),
    out_type=jax.ShapeDtypeStruct((N_NODES // 2, 2 * EMB_DIM), jnp.float32),
    scratch_types=[
        pltpu.VMEM((B_PER_W,), jnp.int32),
        pltpu.VMEM((CHUNK_MAX, EMB_DIM), jnp.float32),
        pltpu.VMEM((CHUNK_MAX, EMB_DIM), jnp.float32),
        pltpu.SemaphoreType.DMA,
        pltpu.SemaphoreType.DMA,
    ],
    compiler_params=pltpu.CompilerParams(use_tc_tiling_on_sc=False),
)
def _sc_gather(idx_hbm, table_hbm, out_hbm, idx_v, buf0, buf1, sem0, sem1):
    _gather_body(idx_hbm, table_hbm, out_hbm, idx_v, buf0, buf1, sem0, sem1)


def kernel(x, table):
    idx = x.reshape(-1)
    slot = ((idx & ~(TBLK - 1)) + ((idx & (TBLK // 2 - 1)) << 1)
            + ((idx >> HALF_SHIFT) & 1))
    slot2 = jnp.concatenate([slot[0::2], slot[1::2]])
    flat = _tc_transpose(table.T)
    tbl = flat.reshape(ROWS_PAD, EMB_DIM)
    out2 = _sc_gather(slot2, tbl)
    return out2.reshape(N_NODES, EMB_DIM)


# speedup vs baseline: 2.7152x; 1.0950x over previous
import functools

import jax
import jax.numpy as jnp
from jax import lax
from jax.experimental import pallas as pl
from jax.experimental.pallas import tpu as pltpu
from jax.experimental.pallas import tpu_sc as plsc

N_NODES = 100000
EMB_DIM = 64
N_ROWS = 1000001

TBLK = 16384
TGRID = -(-N_ROWS // TBLK)
ROWS_PAD = TGRID * TBLK
FLAT = ROWS_PAD * EMB_DIM
HALF_SHIFT = (TBLK // 2).bit_length() - 1

NUM_CORES = 2
NUM_SUBCORES = 16
NW = NUM_CORES * NUM_SUBCORES
B_PER_W = 3136
HB_PER_W = B_PER_W // 2
HCHUNKS = (392, 392, 392, 392)
CHUNK_MAX = 392


def _transpose_body(src_ref, dst_ref):
    x = src_ref[...]
    xx = jnp.concatenate([x[:, :TBLK // 2], x[:, TBLK // 2:]], axis=0)
    dst_ref[...] = xx.T


_tc_transpose = pl.pallas_call(
    _transpose_body,
    grid=(TGRID,),
    in_specs=[pl.BlockSpec((EMB_DIM, TBLK), lambda k: (0, k))],
    out_specs=pl.BlockSpec((TBLK // 2, 2 * EMB_DIM), lambda k: (k, 0)),
    out_shape=jax.ShapeDtypeStruct((FLAT // 128, 128), jnp.float32),
)


def _gather_body(idx_hbm, table_hbm, out_hbm, idx_v, buf0, buf1, sem0, sem1):
    wid = lax.axis_index("s") * NUM_CORES + lax.axis_index("c")
    base = jnp.minimum(wid * B_PER_W, N_NODES - B_PER_W)
    h = pl.multiple_of(base // 2, 8)
    pltpu.sync_copy(idx_hbm.at[pl.ds(h, HB_PER_W)], idx_v.at[pl.ds(0, HB_PER_W)])
    pltpu.sync_copy(idx_hbm.at[pl.ds(N_NODES // 2 + h, HB_PER_W)],
                    idx_v.at[pl.ds(HB_PER_W, HB_PER_W)])
    bufs = (buf0, buf1)
    sems = (sem0, sem1)
    offs = [0]
    for c in HCHUNKS:
        offs.append(offs[-1] + c)
    njobs = 2 * len(HCHUNKS)

    def job(j):
        par = j % 2
        c = j // 2
        return par, offs[c], HCHUNKS[c]

    copies = [None, None]

    def start(j, slot):
        par, off, sz = job(j)
        copies[slot] = pltpu.async_copy(
            table_hbm.at[idx_v.at[pl.ds(par * HB_PER_W + off, sz)]],
            bufs[slot].at[pl.ds(0, sz)], sems[slot])

    start(0, 0)
    for j in range(njobs):
        cur = j % 2
        if j + 1 < njobs:
            start(j + 1, (j + 1) % 2)
        par, off, sz = job(j)
        copies[cur].wait()
        pltpu.sync_copy(
            bufs[cur].at[pl.ds(0, sz)],
            out_hbm.at[pl.ds(h + off, sz),
                       pl.ds(par * EMB_DIM, EMB_DIM)])


@functools.partial(
    pl.kernel,
    mesh=plsc.VectorSubcoreMesh(core_axis_name="c", subcore_axis_name="s"),
    out_type=jax.ShapeDtypeStruct((N_NODES // 2, 2 * EMB_DIM), jnp.float32),
    scratch_types=[
        pltpu.VMEM((B_PER_W,), jnp.int32),
        pltpu.VMEM((CHUNK_MAX, EMB_DIM), jnp.float32),
        pltpu.VMEM((CHUNK_MAX, EMB_DIM), jnp.float32),
        pltpu.SemaphoreType.DMA,
        pltpu.SemaphoreType.DMA,
    ],
    compiler_params=pltpu.CompilerParams(use_tc_tiling_on_sc=False),
)
def _sc_gather(idx_hbm, table_hbm, out_hbm, idx_v, buf0, buf1, sem0, sem1):
    _gather_body(idx_hbm, table_hbm, out_hbm, idx_v, buf0, buf1, sem0, sem1)


def kernel(x, table):
    idx = x.reshape(-1)
    slot = ((idx & ~(TBLK - 1)) + ((idx & (TBLK // 2 - 1)) << 1)
            + ((idx >> HALF_SHIFT) & 1))
    slot2 = jnp.concatenate([slot[0::2], slot[1::2]])
    flat = _tc_transpose(table.T)
    tbl = flat.reshape(ROWS_PAD, EMB_DIM)
    out2 = _sc_gather(slot2, tbl)
    return out2.reshape(N_NODES, EMB_DIM)


# speedup vs baseline: 2.7572x; 1.0155x over previous
import functools

import jax
import jax.numpy as jnp
from jax import lax
from jax.experimental import pallas as pl
from jax.experimental.pallas import tpu as pltpu
from jax.experimental.pallas import tpu_sc as plsc

N_NODES = 100000
EMB_DIM = 64
N_ROWS = 1000001

TBLK = 32768
TGRID = -(-N_ROWS // TBLK)
ROWS_PAD = TGRID * TBLK
FLAT = ROWS_PAD * EMB_DIM
HALF_SHIFT = (TBLK // 2).bit_length() - 1

NUM_CORES = 2
NUM_SUBCORES = 16
NW = NUM_CORES * NUM_SUBCORES
B_PER_W = 3136
HB_PER_W = B_PER_W // 2
HCHUNKS = (392, 392, 392, 392)
CHUNK_MAX = 392


def _transpose_body(src_ref, dst_ref):
    x = src_ref[...]
    xx = jnp.concatenate([x[:, :TBLK // 2], x[:, TBLK // 2:]], axis=0)
    dst_ref[...] = xx.T


_tc_transpose = pl.pallas_call(
    _transpose_body,
    grid=(TGRID,),
    in_specs=[pl.BlockSpec((EMB_DIM, TBLK), lambda k: (0, k))],
    out_specs=pl.BlockSpec((TBLK // 2, 2 * EMB_DIM), lambda k: (k, 0)),
    out_shape=jax.ShapeDtypeStruct((FLAT // 128, 128), jnp.float32),
)


def _gather_body(idx_hbm, table_hbm, out_hbm, idx_v, buf0, buf1, sem0, sem1):
    wid = lax.axis_index("s") * NUM_CORES + lax.axis_index("c")
    base = jnp.minimum(wid * B_PER_W, N_NODES - B_PER_W)
    h = pl.multiple_of(base // 2, 8)
    pltpu.sync_copy(idx_hbm.at[pl.ds(h, HB_PER_W)], idx_v.at[pl.ds(0, HB_PER_W)])
    pltpu.sync_copy(idx_hbm.at[pl.ds(N_NODES // 2 + h, HB_PER_W)],
                    idx_v.at[pl.ds(HB_PER_W, HB_PER_W)])
    bufs = (buf0, buf1)
    sems = (sem0, sem1)
    offs = [0]
    for c in HCHUNKS:
        offs.append(offs[-1] + c)
    njobs = 2 * len(HCHUNKS)

    def job(j):
        par = j % 2
        c = j // 2
        return par, offs[c], HCHUNKS[c]

    copies = [None, None]

    def start(j, slot):
        par, off, sz = job(j)
        copies[slot] = pltpu.async_copy(
            table_hbm.at[idx_v.at[pl.ds(par * HB_PER_W + off, sz)]],
            bufs[slot].at[pl.ds(0, sz)], sems[slot])

    start(0, 0)
    for j in range(njobs):
        cur = j % 2
        if j + 1 < njobs:
            start(j + 1, (j + 1) % 2)
        par, off, sz = job(j)
        copies[cur].wait()
        pltpu.sync_copy(
            bufs[cur].at[pl.ds(0, sz)],
            out_hbm.at[pl.ds(h + off, sz),
                       pl.ds(par * EMB_DIM, EMB_DIM)])


@functools.partial(
    pl.kernel,
    mesh=plsc.VectorSubcoreMesh(core_axis_name="c", subcore_axis_name="s"),
    out_type=jax.ShapeDtypeStruct((N_NODES // 2, 2 * EMB_DIM), jnp.float32),
    scratch_types=[
        pltpu.VMEM((B_PER_W,), jnp.int32),
        pltpu.VMEM((CHUNK_MAX, EMB_DIM), jnp.float32),
        pltpu.VMEM((CHUNK_MAX, EMB_DIM), jnp.float32),
        pltpu.SemaphoreType.DMA,
        pltpu.SemaphoreType.DMA,
    ],
    compiler_params=pltpu.CompilerParams(use_tc_tiling_on_sc=False),
)
def _sc_gather(idx_hbm, table_hbm, out_hbm, idx_v, buf0, buf1, sem0, sem1):
    _gather_body(idx_hbm, table_hbm, out_hbm, idx_v, buf0, buf1, sem0, sem1)


def kernel(x, table):
    idx = x.reshape(-1)
    slot = ((idx & ~(TBLK - 1)) + ((idx & (TBLK // 2 - 1)) << 1)
            + ((idx >> HALF_SHIFT) & 1))
    slot2 = jnp.concatenate([slot[0::2], slot[1::2]])
    flat = _tc_transpose(table.T)
    tbl = flat.reshape(ROWS_PAD, EMB_DIM)
    out2 = _sc_gather(slot2, tbl)
    return out2.reshape(N_NODES, EMB_DIM)
